# baseline (speedup 1.0000x reference)
import functools

import jax
import jax.numpy as jnp
from jax import lax
from jax.experimental import pallas as pl
from jax.experimental.pallas import tpu as pltpu
from jax.experimental.pallas import tpu_sc as plsc

B = 16384
D = 64
H1 = 128

_NC, _NS = 2, 16
_NW = _NC * _NS
_BPW = B // _NW


_CH = 128
_NCH = _BPW // _CH


def _sc_gather_body(uids2, iids2, u_tab, q_tab, a_tab, b_tab,
                    u_out, q_out, ab_out,
                    uidx_v, iidx_v, urows, qrows, a_v, b_v, ab_v, sem):
  wid = lax.axis_index("s") * _NC + lax.axis_index("c")
  base = wid * _BPW
  pltpu.sync_copy(uids2.at[pl.ds(wid * _NCH, _NCH)], uidx_v)
  pltpu.sync_copy(iids2.at[pl.ds(wid * _NCH, _NCH)], iidx_v)
  copies = []
  for j in range(_NCH):
    s = pl.ds(j * _CH, _CH)
    copies.append(pltpu.async_copy(u_tab.at[uidx_v.at[j]], urows.at[s], sem))
    copies.append(pltpu.async_copy(q_tab.at[iidx_v.at[j]], qrows.at[s], sem))
    copies.append(pltpu.async_copy(a_tab.at[uidx_v.at[j]], a_v.at[s], sem))
    copies.append(pltpu.async_copy(b_tab.at[iidx_v.at[j]], b_v.at[s], sem))
  for c in copies:
    c.wait()
  for g in range(_BPW // 16):
    s = pl.ds(g * 16, 16)
    ab_v[s] = a_v[s] + b_v[s]
  pltpu.sync_copy(urows, u_out.at[pl.ds(base, _BPW)])
  pltpu.sync_copy(qrows, q_out.at[pl.ds(base, _BPW)])
  pltpu.sync_copy(ab_v, ab_out.at[pl.ds(base, _BPW)])


@functools.cache
def _make_sc_gather():
  return pl.kernel(
      _sc_gather_body,
      out_type=(
          jax.ShapeDtypeStruct((B, D), jnp.float32),
          jax.ShapeDtypeStruct((B, D), jnp.float32),
          jax.ShapeDtypeStruct((B,), jnp.float32),
      ),
      mesh=plsc.VectorSubcoreMesh(core_axis_name="c", subcore_axis_name="s",
                                  num_cores=_NC, num_subcores=_NS),
      compiler_params=pltpu.CompilerParams(use_tc_tiling_on_sc=False),
      scratch_types=[
          pltpu.VMEM((_NCH, _CH), jnp.int32),
          pltpu.VMEM((_NCH, _CH), jnp.int32),
          pltpu.VMEM((_BPW, D), jnp.float32),
          pltpu.VMEM((_BPW, D), jnp.float32),
          pltpu.VMEM((_BPW,), jnp.float32),
          pltpu.VMEM((_BPW,), jnp.float32),
          pltpu.VMEM((_BPW,), jnp.float32),
          pltpu.SemaphoreType.DMA,
      ],
  )


_BB = 1024
_NB = B // _BB


def _tc_dense_body(u_ref, q_ref, ab_ref, w1t_ref, b1_ref, w2_ref,
                   b2_ref, preds_ref, score_ref):
  u = u_ref[...]
  q = q_ref[...]
  uq = u * q
  preds_ref[...] = jnp.sum(uq, axis=1, keepdims=True) + ab_ref[...]
  w1t = w1t_ref[...]
  h = (jnp.dot(u, w1t[:D], preferred_element_type=jnp.float32)
       + jnp.dot(q, w1t[D:2 * D], preferred_element_type=jnp.float32)
       + jnp.dot(uq, w1t[2 * D:], preferred_element_type=jnp.float32)
       + b1_ref[...])
  h = jnp.maximum(h, 0.0)
  score_ref[...] = (jnp.dot(h, w2_ref[...], preferred_element_type=jnp.float32)
                    + b2_ref[...])


_tc_dense = pl.pallas_call(
    _tc_dense_body,
    grid=(_NB,),
    in_specs=[
        pl.BlockSpec((_BB, D), lambda i: (i, 0)),
        pl.BlockSpec((_BB, D), lambda i: (i, 0)),
        pl.BlockSpec((_BB, 1), lambda i: (i, 0)),
        pl.BlockSpec((3 * D, H1), lambda i: (0, 0)),
        pl.BlockSpec((1, H1), lambda i: (0, 0)),
        pl.BlockSpec((H1, 1), lambda i: (0, 0)),
        pl.BlockSpec((1, 1), lambda i: (0, 0)),
    ],
    out_specs=[
        pl.BlockSpec((_BB, 1), lambda i: (i, 0)),
        pl.BlockSpec((_BB, 1), lambda i: (i, 0)),
    ],
    out_shape=[
        jax.ShapeDtypeStruct((B, 1), jnp.float32),
        jax.ShapeDtypeStruct((B, 1), jnp.float32),
    ],
)


def kernel(user_ids, item_ids, U1, Q1, A1, B1, W1, b1, W2, b2):
  uids = user_ids.astype(jnp.int32)
  iids = item_ids.astype(jnp.int32)
  u, q, ab = _make_sc_gather()(uids.reshape(-1, _CH), iids.reshape(-1, _CH),
                               U1, Q1, A1.reshape(-1), B1.reshape(-1))
  preds, score = _tc_dense(u, q, ab.reshape(B, 1), W1.T, b1.reshape(1, H1),
                           W2.reshape(H1, 1), b2.reshape(1, 1))
  return preds.reshape(-1), score.reshape(-1)


# speedup vs baseline: 1.0532x; 1.0532x over previous
import functools

import jax
import jax.numpy as jnp
from jax import lax
from jax.experimental import pallas as pl
from jax.experimental.pallas import tpu as pltpu
from jax.experimental.pallas import tpu_sc as plsc

B = 16384
D = 64
H1 = 128

_NC, _NS = 2, 16
_NW = _NC * _NS
_BPW = B // _NW


_CH = 128
_NCH = _BPW // _CH


def _sc_gather_body(uids2, iids2, u_tab, q_tab, a_tab, b_tab,
                    u_out, q_out, ab_out,
                    uidx_v, iidx_v, urows, qrows, a_v, b_v, ab_v, sem):
  wid = lax.axis_index("s") * _NC + lax.axis_index("c")
  base = wid * _BPW
  pltpu.sync_copy(uids2.at[pl.ds(wid * _NCH, _NCH)], uidx_v)
  pltpu.sync_copy(iids2.at[pl.ds(wid * _NCH, _NCH)], iidx_v)
  copies = []
  for j in range(_NCH):
    s = pl.ds(j * _CH, _CH)
    copies.append(pltpu.async_copy(u_tab.at[uidx_v.at[j]], urows.at[s], sem))
    copies.append(pltpu.async_copy(q_tab.at[iidx_v.at[j]], qrows.at[s], sem))
    copies.append(pltpu.async_copy(a_tab.at[uidx_v.at[j]], a_v.at[s], sem))
    copies.append(pltpu.async_copy(b_tab.at[iidx_v.at[j]], b_v.at[s], sem))
  for c in copies:
    c.wait()
  for g in range(_BPW // 16):
    s = pl.ds(g * 16, 16)
    ab_v[s] = a_v[s] + b_v[s]
  pltpu.sync_copy(urows, u_out.at[pl.ds(base, _BPW)])
  pltpu.sync_copy(qrows, q_out.at[pl.ds(base, _BPW)])
  pltpu.sync_copy(ab_v, ab_out.at[pl.ds(base, _BPW)])


@functools.cache
def _make_sc_gather():
  return pl.kernel(
      _sc_gather_body,
      out_type=(
          jax.ShapeDtypeStruct((B, D), jnp.float32),
          jax.ShapeDtypeStruct((B, D), jnp.float32),
          jax.ShapeDtypeStruct((B,), jnp.float32),
      ),
      mesh=plsc.VectorSubcoreMesh(core_axis_name="c", subcore_axis_name="s",
                                  num_cores=_NC, num_subcores=_NS),
      compiler_params=pltpu.CompilerParams(use_tc_tiling_on_sc=False),
      scratch_types=[
          pltpu.VMEM((_NCH, _CH), jnp.int32),
          pltpu.VMEM((_NCH, _CH), jnp.int32),
          pltpu.VMEM((_BPW, D), jnp.float32),
          pltpu.VMEM((_BPW, D), jnp.float32),
          pltpu.VMEM((_BPW,), jnp.float32),
          pltpu.VMEM((_BPW,), jnp.float32),
          pltpu.VMEM((_BPW,), jnp.float32),
          pltpu.SemaphoreType.DMA,
      ],
  )


_BB = 1024
_NB = B // _BB


def _tc_dense_body(u_ref, q_ref, ab_ref, w1_ref, b1_ref, w2_ref,
                   b2_ref, preds_ref, score_ref):
  u = u_ref[...]
  q = q_ref[...]
  uq = u * q
  preds_ref[...] = jnp.sum(uq, axis=1) + ab_ref[...]
  w1 = w1_ref[...]
  dn = (((1,), (1,)), ((), ()))
  h = (lax.dot_general(u, w1[:, :D], dn, preferred_element_type=jnp.float32)
       + lax.dot_general(q, w1[:, D:2 * D], dn,
                         preferred_element_type=jnp.float32)
       + lax.dot_general(uq, w1[:, 2 * D:], dn,
                         preferred_element_type=jnp.float32)
       + b1_ref[...][None, :])
  h = jnp.maximum(h, 0.0)
  score_ref[...] = jnp.sum(h * w2_ref[...], axis=1) + b2_ref[0]


_tc_dense = pl.pallas_call(
    _tc_dense_body,
    grid=(_NB,),
    in_specs=[
        pl.BlockSpec((_BB, D), lambda i: (i, 0)),
        pl.BlockSpec((_BB, D), lambda i: (i, 0)),
        pl.BlockSpec((_BB,), lambda i: (i,)),
        pl.BlockSpec((H1, 3 * D), lambda i: (0, 0)),
        pl.BlockSpec((H1,), lambda i: (0,)),
        pl.BlockSpec((1, H1), lambda i: (0, 0)),
        pl.BlockSpec(memory_space=pltpu.SMEM),
    ],
    out_specs=[
        pl.BlockSpec((_BB,), lambda i: (i,)),
        pl.BlockSpec((_BB,), lambda i: (i,)),
    ],
    out_shape=[
        jax.ShapeDtypeStruct((B,), jnp.float32),
        jax.ShapeDtypeStruct((B,), jnp.float32),
    ],
)


def kernel(user_ids, item_ids, U1, Q1, A1, B1, W1, b1, W2, b2):
  uids = user_ids.astype(jnp.int32)
  iids = item_ids.astype(jnp.int32)
  u, q, ab = _make_sc_gather()(uids.reshape(-1, _CH), iids.reshape(-1, _CH),
                               U1, Q1, A1.reshape(-1), B1.reshape(-1))
  preds, score = _tc_dense(u, q, ab, W1, b1, W2, b2)
  return preds, score
